# Initial kernel scaffold; baseline (speedup 1.0000x reference)
import functools

import jax
import jax.numpy as jnp
from jax import lax
from jax.experimental import pallas as pl
from jax.experimental.pallas import tpu as pltpu
from jax.experimental.pallas import tpu_sc as plsc

N_NODES = 10000
N_EDGES = 320000
D_FEAT = 128
D_HID = 16
N_CLASS = 128

NP = 10240
RT = NP // 16
EP = 331776
EROWS = EP // 128
DEG_ROWS = EROWS // 16
CONV_ROWS = EROWS // 32
MB = 512


def _rsqrt_newton(x):
    xi = plsc.bitcast(x, jnp.int32)
    y = plsc.bitcast(jnp.int32(0x5F3759DF) - (xi >> 1), jnp.float32)
    for _ in range(3):
        y = y * (1.5 - 0.5 * x * y * y)
    return y


def _zero_rows(rows_ref):
    z = jnp.zeros((16,), jnp.float32)
    for i in range(128):
        rows_ref[i, :] = z


_MESH = plsc.VectorSubcoreMesh(core_axis_name="c", subcore_axis_name="s")


@functools.partial(
    pl.kernel,
    out_type=(
        jax.ShapeDtypeStruct((2, NP, D_HID), jnp.float32),
        jax.ShapeDtypeStruct((EROWS, 128), jnp.float32),
    ),
    mesh=_MESH,
    scratch_types=(
        pltpu.VMEM_SHARED((NP,), jnp.float32),
        pltpu.VMEM_SHARED((NP, D_HID), jnp.float32),
        pltpu.VMEM_SHARED((NP, D_HID), jnp.float32),
        pltpu.VMEM((DEG_ROWS, 128), jnp.int32),
        pltpu.VMEM((DEG_ROWS, 128), jnp.float32),
        pltpu.VMEM((CONV_ROWS, 128), jnp.int32),
        pltpu.VMEM((CONV_ROWS, 128), jnp.float32),
        pltpu.VMEM((NP,), jnp.float32),
        pltpu.VMEM((RT,), jnp.float32),
        pltpu.VMEM((128, D_HID), jnp.float32),
    ),
)
def _sc_conv1(src_hbm, dst_hbm, ew_hbm, h_hbm, part_hbm, norm_hbm,
              deg_sp, h_sp, agg_sp, dstb, ewb, srcb, normb, dinvl, degl, rows):
    c = lax.axis_index("c")
    s = lax.axis_index("s")
    nbase = s * RT

    _zero_rows(rows)
    for k in range(RT // 128):
        pltpu.sync_copy(rows, agg_sp.at[pl.ds(nbase + 128 * k, 128)])
    z16 = jnp.zeros((16,), jnp.float32)
    for i in range(RT // 16):
        degl[pl.ds(16 * i, 16)] = z16
    pltpu.sync_copy(degl, deg_sp.at[pl.ds(nbase, RT)])
    pltpu.sync_copy(h_hbm.at[pl.ds(nbase, RT)], h_sp.at[pl.ds(nbase, RT)])
    plsc.subcore_barrier()

    dbase = s * DEG_ROWS
    pltpu.sync_copy(dst_hbm.at[pl.ds(dbase, DEG_ROWS)], dstb)
    pltpu.sync_copy(ew_hbm.at[pl.ds(dbase, DEG_ROWS)], ewb)

    def deg_step(i, _):
        pltpu.sync_copy(ewb.at[i], deg_sp.at[dstb.at[i]], add=True)
        return 0

    lax.fori_loop(0, DEG_ROWS, deg_step, 0)
    plsc.subcore_barrier()

    pltpu.sync_copy(deg_sp.at[pl.ds(nbase, RT)], degl)
    for i in range(RT // 16):
        x = jnp.maximum(degl[pl.ds(16 * i, 16)], 1.0)
        degl[pl.ds(16 * i, 16)] = _rsqrt_newton(x)
    pltpu.sync_copy(degl, deg_sp.at[pl.ds(nbase, RT)])
    plsc.subcore_barrier()
    pltpu.sync_copy(deg_sp, dinvl)

    ebase = c * (EROWS // 2) + s * CONV_ROWS
    pltpu.sync_copy(src_hbm.at[pl.ds(ebase, CONV_ROWS)], srcb)
    pltpu.sync_copy(dst_hbm.at[pl.ds(ebase, CONV_ROWS)], dstb.at[pl.ds(0, CONV_ROWS)])
    pltpu.sync_copy(ew_hbm.at[pl.ds(ebase, CONV_ROWS)], ewb.at[pl.ds(0, CONV_ROWS)])

    def conv_step(j, _):
        for g in range(8):
            sl = pl.ds(16 * g, 16)
            sv = srcb[j, sl]
            dv = dstb[j, sl]
            ev = ewb[j, sl]
            nrm = plsc.load_gather(dinvl, [sv]) * ev * plsc.load_gather(dinvl, [dv])
            normb[j, sl] = nrm
        pltpu.sync_copy(h_sp.at[srcb.at[j]], rows)
        for e in range(128):
            rows[e, :] = rows[e, :] * normb[j, e]
        pltpu.sync_copy(rows, agg_sp.at[dstb.at[j]], add=True)
        return 0

    lax.fori_loop(0, CONV_ROWS, conv_step, 0)
    pltpu.sync_copy(normb, norm_hbm.at[pl.ds(ebase, CONV_ROWS)])
    plsc.subcore_barrier()

    pltpu.sync_copy(agg_sp.at[pl.ds(nbase, RT)], part_hbm.at[c, pl.ds(nbase, RT)])


@functools.partial(
    pl.kernel,
    out_type=jax.ShapeDtypeStruct((2, NP, D_HID), jnp.float32),
    mesh=_MESH,
    scratch_types=(
        pltpu.VMEM_SHARED((NP, D_HID), jnp.float32),
        pltpu.VMEM_SHARED((NP, D_HID), jnp.float32),
        pltpu.VMEM((CONV_ROWS, 128), jnp.int32),
        pltpu.VMEM((CONV_ROWS, 128), jnp.int32),
        pltpu.VMEM((CONV_ROWS, 128), jnp.float32),
        pltpu.VMEM((128, D_HID), jnp.float32),
        pltpu.VMEM((RT, D_HID), jnp.float32),
        pltpu.VMEM((RT, D_HID), jnp.float32),
        pltpu.VMEM((16,), jnp.float32),
    ),
)
def _sc_conv2(src_hbm, dst_hbm, norm_hbm, p_hbm, b1_hbm, part_hbm,
              z_sp, agg_sp, srcb, dstb, normb, rows, pbuf0, pbuf1, b1b):
    c = lax.axis_index("c")
    s = lax.axis_index("s")
    nbase = s * RT

    _zero_rows(rows)
    for k in range(RT // 128):
        pltpu.sync_copy(rows, agg_sp.at[pl.ds(nbase + 128 * k, 128)])
    pltpu.sync_copy(p_hbm.at[0, pl.ds(nbase, RT)], pbuf0)
    pltpu.sync_copy(p_hbm.at[1, pl.ds(nbase, RT)], pbuf1)
    pltpu.sync_copy(b1_hbm, b1b)
    b1v = b1b[:]

    def relu_step(i, _):
        pbuf0[i, :] = jnp.maximum(pbuf0[i, :] + pbuf1[i, :] + b1v, 0.0)
        return 0

    lax.fori_loop(0, RT, relu_step, 0)
    pltpu.sync_copy(pbuf0, z_sp.at[pl.ds(nbase, RT)])
    plsc.subcore_barrier()

    ebase = c * (EROWS // 2) + s * CONV_ROWS
    pltpu.sync_copy(src_hbm.at[pl.ds(ebase, CONV_ROWS)], srcb)
    pltpu.sync_copy(dst_hbm.at[pl.ds(ebase, CONV_ROWS)], dstb)
    pltpu.sync_copy(norm_hbm.at[pl.ds(ebase, CONV_ROWS)], normb)

    def conv_step(j, _):
        pltpu.sync_copy(z_sp.at[srcb.at[j]], rows)
        for e in range(128):
            rows[e, :] = rows[e, :] * normb[j, e]
        pltpu.sync_copy(rows, agg_sp.at[dstb.at[j]], add=True)
        return 0

    lax.fori_loop(0, CONV_ROWS, conv_step, 0)
    plsc.subcore_barrier()

    pltpu.sync_copy(agg_sp.at[pl.ds(nbase, RT)], part_hbm.at[c, pl.ds(nbase, RT)])


def _tc_matmul_body(x_ref, w_ref, o_ref):
    o_ref[:, :] = jnp.dot(x_ref[:, :], w_ref[:, :],
                          preferred_element_type=jnp.float32)


def _tc_matmul(x, w):
    return pl.pallas_call(
        _tc_matmul_body,
        grid=(NP // MB,),
        in_specs=[
            pl.BlockSpec((MB, D_FEAT), lambda i: (i, 0)),
            pl.BlockSpec((D_FEAT, D_HID), lambda i: (0, 0)),
        ],
        out_specs=pl.BlockSpec((MB, D_HID), lambda i: (i, 0)),
        out_shape=jax.ShapeDtypeStruct((NP, D_HID), jnp.float32),
    )(x, w)


def _tc_finish_body(q_ref, w_ref, b_ref, o_ref):
    q = q_ref[0] + q_ref[1]
    t = jnp.dot(q, w_ref[:, :], preferred_element_type=jnp.float32) + b_ref[:, :]
    m = jnp.max(t, axis=1, keepdims=True)
    e = jnp.exp(t - m)
    lse = jnp.log(jnp.sum(e, axis=1, keepdims=True))
    o_ref[:, :] = t - m - lse


def _tc_finish(q, w2, b2):
    return pl.pallas_call(
        _tc_finish_body,
        grid=(NP // MB,),
        in_specs=[
            pl.BlockSpec((2, MB, D_HID), lambda i: (0, i, 0)),
            pl.BlockSpec((D_HID, N_CLASS), lambda i: (0, 0)),
            pl.BlockSpec((1, N_CLASS), lambda i: (0, 0)),
        ],
        out_specs=pl.BlockSpec((MB, N_CLASS), lambda i: (i, 0)),
        out_shape=jax.ShapeDtypeStruct((NP, N_CLASS), jnp.float32),
    )(q, w2, b2)


def kernel(x, edge_index, edge_attr, W1, b1, W2, b2):
    n = x.shape[0]
    loop = jnp.arange(n, dtype=jnp.int32)
    src = jnp.concatenate([edge_index[0].astype(jnp.int32), loop])
    dst = jnp.concatenate([edge_index[1].astype(jnp.int32), loop])
    ew = jnp.concatenate([edge_attr, jnp.ones((n,), jnp.float32)])
    pad = EP - src.shape[0]
    src2d = jnp.pad(src, (0, pad)).reshape(EROWS, 128)
    dst2d = jnp.pad(dst, (0, pad)).reshape(EROWS, 128)
    ew2d = jnp.pad(ew, (0, pad)).reshape(EROWS, 128)
    xp = jnp.pad(x, ((0, NP - n), (0, 0)))

    h1 = _tc_matmul(xp, W1)
    part1, norm2d = _sc_conv1(src2d, dst2d, ew2d, h1)
    part2 = _sc_conv2(src2d, dst2d, norm2d, part1, b1)
    out = _tc_finish(part2, W2, b2.reshape(1, N_CLASS))
    return out[:n]


# speedup vs baseline: 61.5003x; 61.5003x over previous
import functools

import jax
import jax.numpy as jnp
from jax import lax
from jax.experimental import pallas as pl
from jax.experimental.pallas import tpu as pltpu
from jax.experimental.pallas import tpu_sc as plsc

N_NODES = 10000
D_FEAT = 128
D_HID = 16
N_CLASS = 128

NP = 10240
RT = NP // 16
EROWS = 2624
EP = EROWS * 128
DEG_ROWS = EROWS // 16
CONV_ROWS = EROWS // 32
MB = 512
FB = 1000


def _rsqrt_newton(x):
    xi = plsc.bitcast(x, jnp.int32)
    y = plsc.bitcast(jnp.int32(0x5F3759DF) - (xi >> 1), jnp.float32)
    for _ in range(3):
        y = y * (1.5 - 0.5 * x * y * y)
    return y


_MESH = plsc.VectorSubcoreMesh(core_axis_name="c", subcore_axis_name="s")


@functools.partial(
    pl.kernel,
    out_type=jax.ShapeDtypeStruct((2, NP, D_HID), jnp.float32),
    mesh=_MESH,
    compiler_params=pltpu.CompilerParams(
        use_tc_tiling_on_sc=False, needs_layout_passes=False),
    scratch_types=(
        pltpu.VMEM_SHARED((NP,), jnp.float32),
        pltpu.VMEM_SHARED((NP, D_HID), jnp.float32),
        pltpu.VMEM_SHARED((NP, D_HID), jnp.float32),
        pltpu.HBM((2, NP, D_HID), jnp.float32),
        pltpu.VMEM((DEG_ROWS, 128), jnp.int32),
        pltpu.VMEM((DEG_ROWS, 128), jnp.float32),
        pltpu.VMEM((CONV_ROWS, 128), jnp.int32),
        pltpu.VMEM((CONV_ROWS, 128), jnp.float32),
        pltpu.VMEM((NP,), jnp.float32),
        pltpu.VMEM((RT,), jnp.float32),
        pltpu.VMEM((128, D_HID), jnp.float32),
        pltpu.VMEM((128, D_HID), jnp.float32),
        pltpu.VMEM((128, D_HID), jnp.float32),
        pltpu.VMEM((RT, D_HID), jnp.float32),
        pltpu.VMEM((RT, D_HID), jnp.float32),
        pltpu.VMEM((16,), jnp.float32),
        pltpu.SemaphoreType.DMA,
        pltpu.SemaphoreType.DMA,
        pltpu.SemaphoreType.DMA,
        pltpu.SemaphoreType.DMA,
        pltpu.SemaphoreType.DMA,
        pltpu.SemaphoreType.REGULAR,
    ),
)
def _sc_gcn(src_hbm, dst_hbm, ew_hbm, h_hbm, b1_hbm, part_hbm,
            deg_sp, tab_sp, agg_sp, p_hbm, dstb, ewb, srcb, normb, dinvl,
            degl, rows0, rows1, zrows, pbuf0, pbuf1, b1b,
            gsem0, gsem1, ssem0, ssem1, dsem, bsem):
    c = lax.axis_index("c")
    s = lax.axis_index("s")
    nbase = s * RT
    nsl = pl.ds(nbase, RT)
    ebase = c * (EROWS // 2) + s * CONV_ROWS

    z16 = jnp.zeros((16,), jnp.float32)

    def zero_agg_slice():
        for k in range(RT // 128):
            pltpu.sync_copy(zrows, agg_sp.at[pl.ds(nbase + 128 * k, 128)])

    def scale_rows(rows, j):
        for g in range(8):
            nv = normb[j, pl.ds(16 * g, 16)]
            for l in range(16):
                e = 16 * g + l
                rows[e, :] = rows[e, :] * nv[l]

    def gather(buf, sem, j):
        return pltpu.async_copy(tab_sp.at[srcb.at[j]], buf, sem)

    def gather_wait(buf, sem, j):
        pltpu.make_async_copy(tab_sp.at[srcb.at[j]], buf, sem).wait()

    def scatter(buf, sem, j):
        return pltpu.async_copy(buf, agg_sp.at[dstb.at[j]], sem, add=True)

    def scatter_wait(buf, sem, j):
        pltpu.make_async_copy(buf, agg_sp.at[dstb.at[j]], sem).wait()

    def conv_pipeline():
        gather(rows0, gsem0, 0)
        gather(rows1, gsem1, 1)

        def pair(jj, _):
            j0 = 2 * jj
            j1 = j0 + 1
            gather_wait(rows0, gsem0, j0)
            scale_rows(rows0, j0)
            scatter(rows0, ssem0, j0)
            gather_wait(rows1, gsem1, j1)
            scale_rows(rows1, j1)
            scatter(rows1, ssem1, j1)
            scatter_wait(rows0, ssem0, j0)
            gather(rows0, gsem0, j0 + 2)
            scatter_wait(rows1, ssem1, j1)
            gather(rows1, gsem1, j1 + 2)
            return 0

        lax.fori_loop(0, CONV_ROWS // 2 - 1, pair, 0)
        j0 = CONV_ROWS - 2
        j1 = CONV_ROWS - 1
        gather_wait(rows0, gsem0, j0)
        scale_rows(rows0, j0)
        scatter(rows0, ssem0, j0)
        gather_wait(rows1, gsem1, j1)
        scale_rows(rows1, j1)
        scatter(rows1, ssem1, j1)
        scatter_wait(rows0, ssem0, j0)
        scatter_wait(rows1, ssem1, j1)

    for i in range(128):
        zrows[i, :] = z16
    zero_agg_slice()
    for i in range(RT // 16):
        degl[pl.ds(16 * i, 16)] = z16
    pltpu.sync_copy(degl, deg_sp.at[nsl])
    pltpu.sync_copy(h_hbm.at[nsl], tab_sp.at[nsl])
    pltpu.sync_copy(b1_hbm, b1b)
    plsc.subcore_barrier()

    dbase = s * DEG_ROWS
    pltpu.sync_copy(dst_hbm.at[pl.ds(dbase, DEG_ROWS)], dstb)
    pltpu.sync_copy(ew_hbm.at[pl.ds(dbase, DEG_ROWS)], ewb)

    def deg_fire(i, _):
        pltpu.async_copy(ewb.at[i], deg_sp.at[dstb.at[i]], dsem, add=True)
        return 0

    def deg_drain(i, _):
        pltpu.make_async_copy(ewb.at[i], deg_sp.at[dstb.at[i]], dsem).wait()
        return 0

    lax.fori_loop(0, DEG_ROWS, deg_fire, 0)
    lax.fori_loop(0, DEG_ROWS, deg_drain, 0)
    plsc.subcore_barrier()

    pltpu.sync_copy(deg_sp.at[nsl], degl)
    for i in range(RT // 16):
        x = jnp.maximum(degl[pl.ds(16 * i, 16)], 1.0)
        degl[pl.ds(16 * i, 16)] = _rsqrt_newton(x)
    pltpu.sync_copy(degl, deg_sp.at[nsl])
    plsc.subcore_barrier()
    pltpu.sync_copy(deg_sp, dinvl)

    pltpu.sync_copy(src_hbm.at[pl.ds(ebase, CONV_ROWS)], srcb)
    pltpu.sync_copy(dst_hbm.at[pl.ds(ebase, CONV_ROWS)], dstb.at[pl.ds(0, CONV_ROWS)])
    pltpu.sync_copy(ew_hbm.at[pl.ds(ebase, CONV_ROWS)], ewb.at[pl.ds(0, CONV_ROWS)])

    def norm_row(j, _):
        for g in range(8):
            sl = pl.ds(16 * g, 16)
            nrm = (plsc.load_gather(dinvl, [srcb[j, sl]]) * ewb[j, sl]
                   * plsc.load_gather(dinvl, [dstb[j, sl]]))
            normb[j, sl] = nrm
        return 0

    lax.fori_loop(0, CONV_ROWS, norm_row, 0)
    conv_pipeline()

    plsc.subcore_barrier()
    pltpu.sync_copy(agg_sp.at[nsl], p_hbm.at[c, nsl])
    zero_agg_slice()
    plsc.subcore_barrier()
    pltpu.core_barrier(bsem, core_axis_name="c")

    pltpu.sync_copy(p_hbm.at[0, nsl], pbuf0)
    pltpu.sync_copy(p_hbm.at[1, nsl], pbuf1)
    b1v = b1b[:]

    def relu_step(i, _):
        pbuf0[i, :] = jnp.maximum(pbuf0[i, :] + pbuf1[i, :] + b1v, 0.0)
        return 0

    lax.fori_loop(0, RT, relu_step, 0)
    pltpu.sync_copy(pbuf0, tab_sp.at[nsl])
    plsc.subcore_barrier()

    conv_pipeline()

    plsc.subcore_barrier()
    pltpu.sync_copy(agg_sp.at[nsl], part_hbm.at[c, nsl])


def _tc_matmul_body(x_ref, w_ref, o_ref):
    o_ref[:, :] = jnp.dot(x_ref[:, :], w_ref[:, :],
                          preferred_element_type=jnp.float32)


def _tc_matmul(x, w):
    return pl.pallas_call(
        _tc_matmul_body,
        grid=(NP // MB,),
        in_specs=[
            pl.BlockSpec((MB, D_FEAT), lambda i: (i, 0)),
            pl.BlockSpec((D_FEAT, D_HID), lambda i: (0, 0)),
        ],
        out_specs=pl.BlockSpec((MB, D_HID), lambda i: (i, 0)),
        out_shape=jax.ShapeDtypeStruct((NP, D_HID), jnp.float32),
    )(x, w)


def _tc_finish_body(q_ref, w_ref, b_ref, o_ref):
    q = q_ref[0] + q_ref[1]
    t = jnp.dot(q, w_ref[:, :], preferred_element_type=jnp.float32) + b_ref[:, :]
    m = jnp.max(t, axis=1, keepdims=True)
    e = jnp.exp(t - m)
    lse = jnp.log(jnp.sum(e, axis=1, keepdims=True))
    o_ref[:, :] = t - m - lse


def _tc_finish(q, w2, b2):
    return pl.pallas_call(
        _tc_finish_body,
        grid=(N_NODES // FB,),
        in_specs=[
            pl.BlockSpec((2, FB, D_HID), lambda i: (0, i, 0)),
            pl.BlockSpec((D_HID, N_CLASS), lambda i: (0, 0)),
            pl.BlockSpec((1, N_CLASS), lambda i: (0, 0)),
        ],
        out_specs=pl.BlockSpec((FB, N_CLASS), lambda i: (i, 0)),
        out_shape=jax.ShapeDtypeStruct((N_NODES, N_CLASS), jnp.float32),
    )(q, w2, b2)


def kernel(x, edge_index, edge_attr, W1, b1, W2, b2):
    n = x.shape[0]
    loop = jnp.arange(n, dtype=jnp.int32)
    src = jnp.concatenate([edge_index[0].astype(jnp.int32), loop])
    dst = jnp.concatenate([edge_index[1].astype(jnp.int32), loop])
    ew = jnp.concatenate([edge_attr, jnp.ones((n,), jnp.float32)])
    pad = EP - src.shape[0]
    src2d = jnp.pad(src, (0, pad)).reshape(EROWS, 128)
    dst2d = jnp.pad(dst, (0, pad)).reshape(EROWS, 128)
    ew2d = jnp.pad(ew, (0, pad)).reshape(EROWS, 128)
    xp = jnp.pad(x, ((0, NP - n), (0, 0)))

    h1 = _tc_matmul(xp, W1)
    part2 = _sc_gcn(src2d, dst2d, ew2d, h1, b1)
    return _tc_finish(part2, W2, b2.reshape(1, N_CLASS))


# speedup vs baseline: 73.5862x; 1.1965x over previous
import functools

import jax
import jax.numpy as jnp
from jax import lax
from jax.experimental import pallas as pl
from jax.experimental.pallas import tpu as pltpu
from jax.experimental.pallas import tpu_sc as plsc

N_NODES = 10000
D_FEAT = 128
D_HID = 16
N_CLASS = 128

NP = 10240
RT = NP // 16
LASTT = N_NODES - 15 * RT
EROWS = 2500
DEG_ROWS = 158
DEG_LAST = EROWS - 15 * DEG_ROWS
CONV_ROWS = 79
CONV_PAIRS = CONV_ROWS // 2 - 1
SLC = RT // 128
MB = 1000
FB = 1000


def _rsqrt_newton(x):
    xi = plsc.bitcast(x, jnp.int32)
    y = plsc.bitcast(jnp.int32(0x5F3759DF) - (xi >> 1), jnp.float32)
    for _ in range(3):
        y = y * (1.5 - 0.5 * x * y * y)
    return y


_MESH = plsc.VectorSubcoreMesh(core_axis_name="c", subcore_axis_name="s")


@functools.partial(
    pl.kernel,
    out_type=jax.ShapeDtypeStruct((2, NP, D_HID), jnp.float32),
    mesh=_MESH,
    compiler_params=pltpu.CompilerParams(
        use_tc_tiling_on_sc=False, needs_layout_passes=False),
    scratch_types=(
        pltpu.VMEM_SHARED((NP,), jnp.float32),
        pltpu.VMEM_SHARED((NP, D_HID), jnp.float32),
        pltpu.VMEM_SHARED((NP, D_HID), jnp.float32),
        pltpu.HBM((2, NP, D_HID), jnp.float32),
        pltpu.VMEM((DEG_ROWS, 128), jnp.int32),
        pltpu.VMEM((DEG_ROWS, 128), jnp.float32),
        pltpu.VMEM((CONV_ROWS, 128), jnp.int32),
        pltpu.VMEM((CONV_ROWS, 128), jnp.float32),
        pltpu.VMEM((SLC, 128), jnp.int32),
        pltpu.VMEM((NP,), jnp.float32),
        pltpu.VMEM((RT,), jnp.float32),
        pltpu.VMEM((128, D_HID), jnp.float32),
        pltpu.VMEM((128, D_HID), jnp.float32),
        pltpu.VMEM((128, D_HID), jnp.float32),
        pltpu.VMEM((RT, D_HID), jnp.float32),
        pltpu.VMEM((RT, D_HID), jnp.float32),
        pltpu.VMEM((16,), jnp.float32),
        pltpu.SemaphoreType.DMA,
        pltpu.SemaphoreType.DMA,
        pltpu.SemaphoreType.DMA,
        pltpu.SemaphoreType.DMA,
        pltpu.SemaphoreType.DMA,
        pltpu.SemaphoreType.REGULAR,
    ),
)
def _sc_gcn(ei_hbm, ew2_hbm, h_hbm, b1_hbm, part_hbm,
            deg_sp, tab_sp, agg_sp, p_hbm, dstb, ewb, srcb, normb, idxb,
            dinvl, degl, rows0, rows1, zrows, pbuf0, pbuf1, b1b,
            gsem0, gsem1, ssem0, ssem1, dsem, bsem):
    c = lax.axis_index("c")
    s = lax.axis_index("s")
    w = c * 16 + s
    nbase = s * RT
    nsl = pl.ds(nbase, RT)

    z16 = jnp.zeros((16,), jnp.float32)
    one16 = jnp.full((16,), 1.0, jnp.float32)

    def zero_agg_slice():
        for k in range(SLC):
            pltpu.sync_copy(zrows, agg_sp.at[pl.ds(nbase + 128 * k, 128)])

    def scale_rows(rows, j):
        for g in range(8):
            nv = normb[j, pl.ds(16 * g, 16)]
            for l in range(16):
                e = 16 * g + l
                rows[e, :] = rows[e, :] * nv[l]

    def gather(buf, sem, j):
        return pltpu.async_copy(tab_sp.at[srcb.at[j]], buf, sem)

    def gather_wait(buf, sem, j):
        pltpu.make_async_copy(tab_sp.at[srcb.at[j]], buf, sem).wait()

    def scatter(buf, sem, j):
        return pltpu.async_copy(buf, agg_sp.at[dstb.at[j]], sem, add=True)

    def scatter_wait(buf, sem, j):
        pltpu.make_async_copy(buf, agg_sp.at[dstb.at[j]], sem).wait()

    def conv_pipeline():
        gather(rows0, gsem0, 0)
        gather(rows1, gsem1, 1)

        def pair(jj, _):
            j0 = 2 * jj
            j1 = j0 + 1
            gather_wait(rows0, gsem0, j0)
            scale_rows(rows0, j0)
            scatter(rows0, ssem0, j0)
            gather_wait(rows1, gsem1, j1)
            scale_rows(rows1, j1)
            scatter(rows1, ssem1, j1)
            scatter_wait(rows0, ssem0, j0)
            gather(rows0, gsem0, j0 + 2)
            scatter_wait(rows1, ssem1, j1)
            gather(rows1, gsem1, j1 + 2)
            return 0

        lax.fori_loop(0, CONV_PAIRS, pair, 0)
        j0 = CONV_ROWS - 3
        j1 = CONV_ROWS - 2
        jt = CONV_ROWS - 1
        gather_wait(rows0, gsem0, j0)
        scale_rows(rows0, j0)
        scatter(rows0, ssem0, j0)
        gather_wait(rows1, gsem1, j1)
        scale_rows(rows1, j1)
        scatter(rows1, ssem1, j1)
        scatter_wait(rows0, ssem0, j0)
        gather(rows0, gsem0, jt)
        gather_wait(rows0, gsem0, jt)
        scale_rows(rows0, jt)
        scatter(rows0, ssem0, jt)
        scatter_wait(rows0, ssem0, jt)
        scatter_wait(rows1, ssem1, j1)

    def selfloop_add():
        for k in range(SLC):
            pltpu.sync_copy(tab_sp.at[pl.ds(nbase + 128 * k, 128)], rows0)
            for g in range(8):
                dv = dinvl[pl.ds(nbase + 128 * k + 16 * g, 16)]
                dv2 = dv * dv
                for l in range(16):
                    e = 16 * g + l
                    rows0[e, :] = rows0[e, :] * dv2[l]
            pltpu.sync_copy(rows0, agg_sp.at[idxb.at[k]], add=True)

    for i in range(128):
        zrows[i, :] = z16
    zero_agg_slice()
    for i in range(RT // 16):
        degl[pl.ds(16 * i, 16)] = one16
    pltpu.sync_copy(degl, deg_sp.at[nsl])
    lanes = lax.iota(jnp.int32, 16)
    for k in range(SLC):
        for g in range(8):
            idxb[k, pl.ds(16 * g, 16)] = nbase + 128 * k + 16 * g + lanes

    @pl.when(s == 15)
    def _():
        pltpu.sync_copy(h_hbm.at[pl.ds(15 * RT, LASTT)],
                        tab_sp.at[pl.ds(15 * RT, LASTT)])

    @pl.when(s < 15)
    def _():
        pltpu.sync_copy(h_hbm.at[nsl], tab_sp.at[nsl])

    pltpu.sync_copy(b1_hbm, b1b)
    plsc.subcore_barrier()

    dstart = s * DEG_ROWS
    nrows_deg = jnp.where(s == 15, DEG_LAST, DEG_ROWS)

    @pl.when(s == 15)
    def _():
        pltpu.sync_copy(ei_hbm.at[1, pl.ds(15 * DEG_ROWS, DEG_LAST)],
                        dstb.at[pl.ds(0, DEG_LAST)])
        pltpu.sync_copy(ew2_hbm.at[pl.ds(15 * DEG_ROWS, DEG_LAST)],
                        ewb.at[pl.ds(0, DEG_LAST)])

    @pl.when(s < 15)
    def _():
        pltpu.sync_copy(ei_hbm.at[1, pl.ds(dstart, DEG_ROWS)], dstb)
        pltpu.sync_copy(ew2_hbm.at[pl.ds(dstart, DEG_ROWS)], ewb)

    def deg_fire(i, _):
        pltpu.async_copy(ewb.at[i], deg_sp.at[dstb.at[i]], dsem, add=True)
        return 0

    def deg_drain(i, _):
        pltpu.make_async_copy(ewb.at[i], deg_sp.at[dstb.at[i]], dsem).wait()
        return 0

    lax.fori_loop(0, nrows_deg, deg_fire, 0)
    lax.fori_loop(0, nrows_deg, deg_drain, 0)
    plsc.subcore_barrier()

    pltpu.sync_copy(deg_sp.at[nsl], degl)
    for i in range(RT // 16):
        x = jnp.maximum(degl[pl.ds(16 * i, 16)], 1.0)
        degl[pl.ds(16 * i, 16)] = _rsqrt_newton(x)
    pltpu.sync_copy(degl, deg_sp.at[nsl])
    plsc.subcore_barrier()
    pltpu.sync_copy(deg_sp, dinvl)

    estart = jnp.where(w == 31, EROWS - CONV_ROWS, w * CONV_ROWS)
    esl = pl.ds(estart, CONV_ROWS)
    pltpu.sync_copy(ei_hbm.at[0, esl], srcb)
    pltpu.sync_copy(ei_hbm.at[1, esl], dstb.at[pl.ds(0, CONV_ROWS)])
    pltpu.sync_copy(ew2_hbm.at[esl], ewb.at[pl.ds(0, CONV_ROWS)])

    def norm_row(j, _):
        for g in range(8):
            sl = pl.ds(16 * g, 16)
            nrm = (plsc.load_gather(dinvl, [srcb[j, sl]]) * ewb[j, sl]
                   * plsc.load_gather(dinvl, [dstb[j, sl]]))
            normb[j, sl] = nrm
        return 0

    lax.fori_loop(0, CONV_ROWS, norm_row, 0)

    @pl.when(w == 30)
    def _():
        for j in range((EROWS - CONV_ROWS) - 30 * CONV_ROWS, CONV_ROWS):
            for g in range(8):
                normb[j, pl.ds(16 * g, 16)] = z16

    conv_pipeline()

    @pl.when(c == 0)
    def _():
        selfloop_add()

    plsc.subcore_barrier()
    pltpu.sync_copy(agg_sp.at[nsl], p_hbm.at[c, nsl])
    zero_agg_slice()
    plsc.subcore_barrier()
    pltpu.core_barrier(bsem, core_axis_name="c")

    pltpu.sync_copy(p_hbm.at[0, nsl], pbuf0)
    pltpu.sync_copy(p_hbm.at[1, nsl], pbuf1)
    b1v = b1b[:]

    def relu_step(i, _):
        pbuf0[i, :] = jnp.maximum(pbuf0[i, :] + pbuf1[i, :] + b1v, 0.0)
        return 0

    lax.fori_loop(0, RT, relu_step, 0)
    pltpu.sync_copy(pbuf0, tab_sp.at[nsl])
    plsc.subcore_barrier()

    conv_pipeline()

    @pl.when(c == 0)
    def _():
        selfloop_add()

    plsc.subcore_barrier()
    pltpu.sync_copy(agg_sp.at[nsl], part_hbm.at[c, nsl])


def _tc_matmul_body(x_ref, w_ref, o_ref):
    o_ref[:, :] = jnp.dot(x_ref[:, :], w_ref[:, :],
                          preferred_element_type=jnp.float32)


def _tc_matmul(x, w):
    return pl.pallas_call(
        _tc_matmul_body,
        grid=(N_NODES // MB,),
        in_specs=[
            pl.BlockSpec((MB, D_FEAT), lambda i: (i, 0)),
            pl.BlockSpec((D_FEAT, D_HID), lambda i: (0, 0)),
        ],
        out_specs=pl.BlockSpec((MB, D_HID), lambda i: (i, 0)),
        out_shape=jax.ShapeDtypeStruct((N_NODES, D_HID), jnp.float32),
    )(x, w)


def _tc_finish_body(q_ref, w_ref, b_ref, o_ref):
    q = q_ref[0] + q_ref[1]
    t = jnp.dot(q, w_ref[:, :], preferred_element_type=jnp.float32) + b_ref[:, :]
    m = jnp.max(t, axis=1, keepdims=True)
    e = jnp.exp(t - m)
    lse = jnp.log(jnp.sum(e, axis=1, keepdims=True))
    o_ref[:, :] = t - m - lse


def _tc_finish(q, w2, b2):
    return pl.pallas_call(
        _tc_finish_body,
        grid=(N_NODES // FB,),
        in_specs=[
            pl.BlockSpec((2, FB, D_HID), lambda i: (0, i, 0)),
            pl.BlockSpec((D_HID, N_CLASS), lambda i: (0, 0)),
            pl.BlockSpec((1, N_CLASS), lambda i: (0, 0)),
        ],
        out_specs=pl.BlockSpec((FB, N_CLASS), lambda i: (i, 0)),
        out_shape=jax.ShapeDtypeStruct((N_NODES, N_CLASS), jnp.float32),
    )(q, w2, b2)


def kernel(x, edge_index, edge_attr, W1, b1, W2, b2):
    ei3d = edge_index.astype(jnp.int32).reshape(2, EROWS, 128)
    ew2d = edge_attr.reshape(EROWS, 128)
    h1 = _tc_matmul(x, W1)
    part2 = _sc_gcn(ei3d, ew2d, h1, b1)
    return _tc_finish(part2, W2, b2.reshape(1, N_CLASS))


# speedup vs baseline: 74.2853x; 1.0095x over previous
import functools

import jax
import jax.numpy as jnp
from jax import lax
from jax.experimental import pallas as pl
from jax.experimental.pallas import tpu as pltpu
from jax.experimental.pallas import tpu_sc as plsc

N_NODES = 10000
D_FEAT = 128
D_HID = 16
N_CLASS = 128

NP = 10240
RT = NP // 16
LASTT = N_NODES - 15 * RT
EROWS = 2500
DEG_ROWS = 79
DEG_LAST = EROWS // 2 - 15 * DEG_ROWS
CONV_ROWS = 80
QUADS = CONV_ROWS // 4
NBUF = 4
SLC = RT // 128
MB = 1000
FB = 1000


def _rsqrt_newton(x):
    xi = plsc.bitcast(x, jnp.int32)
    y = plsc.bitcast(jnp.int32(0x5F3759DF) - (xi >> 1), jnp.float32)
    for _ in range(3):
        y = y * (1.5 - 0.5 * x * y * y)
    return y


_MESH = plsc.VectorSubcoreMesh(core_axis_name="c", subcore_axis_name="s")


@functools.partial(
    pl.kernel,
    out_type=jax.ShapeDtypeStruct((2, NP, D_HID), jnp.float32),
    mesh=_MESH,
    compiler_params=pltpu.CompilerParams(
        use_tc_tiling_on_sc=False, needs_layout_passes=False),
    scratch_types=(
        pltpu.VMEM_SHARED((NP,), jnp.float32),
        pltpu.VMEM_SHARED((NP, D_HID), jnp.float32),
        pltpu.VMEM_SHARED((NP, D_HID), jnp.float32),
        pltpu.HBM((2, NP, D_HID), jnp.float32),
        pltpu.HBM((2, NP), jnp.float32),
        pltpu.VMEM((CONV_ROWS, 128), jnp.int32),
        pltpu.VMEM((CONV_ROWS, 128), jnp.float32),
        pltpu.VMEM((CONV_ROWS, 128), jnp.int32),
        pltpu.VMEM((CONV_ROWS, 128), jnp.float32),
        pltpu.VMEM((SLC, 128), jnp.int32),
        pltpu.VMEM((NP,), jnp.float32),
        pltpu.VMEM((RT,), jnp.float32),
        pltpu.VMEM((RT,), jnp.float32),
        pltpu.VMEM((128, D_HID), jnp.float32),
        pltpu.VMEM((128, D_HID), jnp.float32),
        pltpu.VMEM((128, D_HID), jnp.float32),
        pltpu.VMEM((128, D_HID), jnp.float32),
        pltpu.VMEM((128, D_HID), jnp.float32),
        pltpu.VMEM((RT, D_HID), jnp.float32),
        pltpu.VMEM((RT, D_HID), jnp.float32),
        pltpu.VMEM((16,), jnp.float32),
        pltpu.SemaphoreType.DMA,
        pltpu.SemaphoreType.DMA,
        pltpu.SemaphoreType.DMA,
        pltpu.SemaphoreType.DMA,
        pltpu.SemaphoreType.DMA,
        pltpu.SemaphoreType.DMA,
        pltpu.SemaphoreType.DMA,
        pltpu.SemaphoreType.DMA,
        pltpu.SemaphoreType.DMA,
        pltpu.SemaphoreType.REGULAR,
    ),
)
def _sc_gcn(ei_hbm, ew2_hbm, h_hbm, b1_hbm, part_hbm,
            deg_sp, tab_sp, agg_sp, p_hbm, dg_hbm, dstb, ewb, srcb, normb,
            idxb, dinvl, degl, degl2, rows0, rows1, rows2, rows3, zrows,
            pbuf0, pbuf1, b1b,
            gsem0, gsem1, gsem2, gsem3, ssem0, ssem1, ssem2, ssem3,
            dsem, bsem):
    c = lax.axis_index("c")
    s = lax.axis_index("s")
    w = c * 16 + s
    nbase = s * RT
    nsl = pl.ds(nbase, RT)

    bufs = (rows0, rows1, rows2, rows3)
    gsems = (gsem0, gsem1, gsem2, gsem3)
    ssems = (ssem0, ssem1, ssem2, ssem3)

    z16 = jnp.zeros((16,), jnp.float32)

    def zero_agg_slice():
        for k in range(SLC):
            pltpu.sync_copy(zrows, agg_sp.at[pl.ds(nbase + 128 * k, 128)])

    def scale_rows(rows, j):
        for g in range(8):
            nv = normb[j, pl.ds(16 * g, 16)]
            for l in range(16):
                e = 16 * g + l
                rows[e, :] = rows[e, :] * nv[l]

    def gather(b, j):
        pltpu.async_copy(tab_sp.at[srcb.at[j]], bufs[b], gsems[b])

    def gather_wait(b, j):
        pltpu.make_async_copy(tab_sp.at[srcb.at[j]], bufs[b], gsems[b]).wait()

    def scatter(b, j):
        pltpu.async_copy(bufs[b], agg_sp.at[dstb.at[j]], ssems[b], add=True)

    def scatter_wait(b, j):
        pltpu.make_async_copy(bufs[b], agg_sp.at[dstb.at[j]], ssems[b]).wait()

    def conv_pipeline():
        for b in range(NBUF):
            gather(b, b)

        def quad(jj, _):
            j = NBUF * jj
            for b in range(NBUF):
                gather_wait(b, j + b)
                scale_rows(bufs[b], j + b)
                scatter(b, j + b)
            for b in range(NBUF):
                scatter_wait(b, j + b)
                gather(b, j + b + NBUF)
            return 0

        lax.fori_loop(0, QUADS - 1, quad, 0)
        j = CONV_ROWS - NBUF
        for b in range(NBUF):
            gather_wait(b, j + b)
            scale_rows(bufs[b], j + b)
            scatter(b, j + b)
        for b in range(NBUF):
            scatter_wait(b, j + b)

    def selfloop_add():
        for k in range(SLC):
            pltpu.sync_copy(tab_sp.at[pl.ds(nbase + 128 * k, 128)], rows0)
            for g in range(8):
                dv = dinvl[pl.ds(nbase + 128 * k + 16 * g, 16)]
                dv2 = dv * dv
                for l in range(16):
                    e = 16 * g + l
                    rows0[e, :] = rows0[e, :] * dv2[l]
            pltpu.sync_copy(rows0, agg_sp.at[idxb.at[k]], add=True)

    for i in range(128):
        zrows[i, :] = z16
    zero_agg_slice()
    selfw = jnp.where(c == 0, jnp.float32(1.0), jnp.float32(0.0))
    initv = jnp.full((16,), 1.0, jnp.float32) * selfw
    for i in range(RT // 16):
        degl[pl.ds(16 * i, 16)] = initv
    pltpu.sync_copy(degl, deg_sp.at[nsl])
    lanes = lax.iota(jnp.int32, 16)
    for k in range(SLC):
        for g in range(8):
            idxb[k, pl.ds(16 * g, 16)] = nbase + 128 * k + 16 * g + lanes

    @pl.when(s == 15)
    def _():
        pltpu.sync_copy(h_hbm.at[pl.ds(15 * RT, LASTT)],
                        tab_sp.at[pl.ds(15 * RT, LASTT)])

    @pl.when(s < 15)
    def _():
        pltpu.sync_copy(h_hbm.at[nsl], tab_sp.at[nsl])

    pltpu.sync_copy(b1_hbm, b1b)
    plsc.subcore_barrier()

    dstart = c * (EROWS // 2) + s * DEG_ROWS
    nrows_deg = jnp.where(s == 15, DEG_LAST, DEG_ROWS)

    @pl.when(s == 15)
    def _():
        pltpu.sync_copy(ei_hbm.at[1, pl.ds(c * (EROWS // 2) + 15 * DEG_ROWS, DEG_LAST)],
                        dstb.at[pl.ds(0, DEG_LAST)])
        pltpu.sync_copy(ew2_hbm.at[pl.ds(c * (EROWS // 2) + 15 * DEG_ROWS, DEG_LAST)],
                        ewb.at[pl.ds(0, DEG_LAST)])

    @pl.when(s < 15)
    def _():
        pltpu.sync_copy(ei_hbm.at[1, pl.ds(dstart, DEG_ROWS)],
                        dstb.at[pl.ds(0, DEG_ROWS)])
        pltpu.sync_copy(ew2_hbm.at[pl.ds(dstart, DEG_ROWS)],
                        ewb.at[pl.ds(0, DEG_ROWS)])

    def deg_fire(i, _):
        pltpu.async_copy(ewb.at[i], deg_sp.at[dstb.at[i]], dsem, add=True)
        return 0

    def deg_drain(i, _):
        pltpu.make_async_copy(ewb.at[i], deg_sp.at[dstb.at[i]], dsem).wait()
        return 0

    lax.fori_loop(0, nrows_deg, deg_fire, 0)
    lax.fori_loop(0, nrows_deg, deg_drain, 0)
    plsc.subcore_barrier()

    pltpu.sync_copy(deg_sp.at[nsl], degl)
    pltpu.sync_copy(degl, dg_hbm.at[c, nsl])
    plsc.subcore_barrier()
    pltpu.core_barrier(bsem, core_axis_name="c")
    pltpu.sync_copy(dg_hbm.at[1 - c, nsl], degl2)
    for i in range(RT // 16):
        sl = pl.ds(16 * i, 16)
        x = jnp.maximum(degl[sl] + degl2[sl], 1.0)
        degl[sl] = _rsqrt_newton(x)
    pltpu.sync_copy(degl, deg_sp.at[nsl])
    plsc.subcore_barrier()
    pltpu.sync_copy(deg_sp, dinvl)

    estart = jnp.where(w == 31, EROWS - CONV_ROWS, w * CONV_ROWS)
    esl = pl.ds(estart, CONV_ROWS)
    pltpu.sync_copy(ei_hbm.at[0, esl], srcb)
    pltpu.sync_copy(ei_hbm.at[1, esl], dstb)
    pltpu.sync_copy(ew2_hbm.at[esl], ewb)

    def norm_row(j, _):
        for g in range(8):
            sl = pl.ds(16 * g, 16)
            nrm = (plsc.load_gather(dinvl, [srcb[j, sl]]) * ewb[j, sl]
                   * plsc.load_gather(dinvl, [dstb[j, sl]]))
            normb[j, sl] = nrm
        return 0

    lax.fori_loop(0, CONV_ROWS, norm_row, 0)

    @pl.when(w == 30)
    def _():
        for j in range((EROWS - CONV_ROWS) - 30 * CONV_ROWS, CONV_ROWS):
            for g in range(8):
                normb[j, pl.ds(16 * g, 16)] = z16

    conv_pipeline()

    @pl.when(c == 0)
    def _():
        selfloop_add()

    plsc.subcore_barrier()
    pltpu.sync_copy(agg_sp.at[nsl], p_hbm.at[c, nsl])
    zero_agg_slice()
    plsc.subcore_barrier()
    pltpu.core_barrier(bsem, core_axis_name="c")

    pltpu.sync_copy(p_hbm.at[0, nsl], pbuf0)
    pltpu.sync_copy(p_hbm.at[1, nsl], pbuf1)
    b1v = b1b[:]

    def relu_step(ii, _):
        for r in range(16):
            i = 16 * ii + r
            pbuf0[i, :] = jnp.maximum(pbuf0[i, :] + pbuf1[i, :] + b1v, 0.0)
        return 0

    lax.fori_loop(0, RT // 16, relu_step, 0)
    pltpu.sync_copy(pbuf0, tab_sp.at[nsl])
    plsc.subcore_barrier()

    conv_pipeline()

    @pl.when(c == 0)
    def _():
        selfloop_add()

    plsc.subcore_barrier()
    pltpu.sync_copy(agg_sp.at[nsl], part_hbm.at[c, nsl])


def _tc_matmul_body(x_ref, w_ref, o_ref):
    o_ref[:, :] = jnp.dot(x_ref[:, :], w_ref[:, :],
                          preferred_element_type=jnp.float32)


def _tc_matmul(x, w):
    return pl.pallas_call(
        _tc_matmul_body,
        grid=(N_NODES // MB,),
        in_specs=[
            pl.BlockSpec((MB, D_FEAT), lambda i: (i, 0)),
            pl.BlockSpec((D_FEAT, D_HID), lambda i: (0, 0)),
        ],
        out_specs=pl.BlockSpec((MB, D_HID), lambda i: (i, 0)),
        out_shape=jax.ShapeDtypeStruct((N_NODES, D_HID), jnp.float32),
    )(x, w)


def _tc_finish_body(q_ref, w_ref, b_ref, o_ref):
    q = q_ref[0] + q_ref[1]
    t = jnp.dot(q, w_ref[:, :], preferred_element_type=jnp.float32) + b_ref[:, :]
    m = jnp.max(t, axis=1, keepdims=True)
    e = jnp.exp(t - m)
    lse = jnp.log(jnp.sum(e, axis=1, keepdims=True))
    o_ref[:, :] = t - m - lse


def _tc_finish(q, w2, b2):
    return pl.pallas_call(
        _tc_finish_body,
        grid=(N_NODES // FB,),
        in_specs=[
            pl.BlockSpec((2, FB, D_HID), lambda i: (0, i, 0)),
            pl.BlockSpec((D_HID, N_CLASS), lambda i: (0, 0)),
            pl.BlockSpec((1, N_CLASS), lambda i: (0, 0)),
        ],
        out_specs=pl.BlockSpec((FB, N_CLASS), lambda i: (i, 0)),
        out_shape=jax.ShapeDtypeStruct((N_NODES, N_CLASS), jnp.float32),
    )(q, w2, b2)


def kernel(x, edge_index, edge_attr, W1, b1, W2, b2):
    ei3d = edge_index.astype(jnp.int32).reshape(2, EROWS, 128)
    ew2d = edge_attr.reshape(EROWS, 128)
    h1 = _tc_matmul(x, W1)
    part2 = _sc_gcn(ei3d, ew2d, h1, b1)
    return _tc_finish(part2, W2, b2.reshape(1, N_CLASS))


# speedup vs baseline: 76.7948x; 1.0338x over previous
import functools

import jax
import jax.numpy as jnp
from jax import lax
from jax.experimental import pallas as pl
from jax.experimental.pallas import tpu as pltpu
from jax.experimental.pallas import tpu_sc as plsc

N_NODES = 10000
D_FEAT = 128
D_HID = 16
N_CLASS = 128

NP = 10240
RT = NP // 16
LASTT = N_NODES - 15 * RT
EROWS = 2500
DEG_ROWS = 79
DEG_LAST = EROWS // 2 - 15 * DEG_ROWS
CONV_ROWS = 80
QUADS = CONV_ROWS // 4
NBUF = 4
SLC = RT // 128
MB = 2000
FB = 2000


def _rsqrt_newton(x):
    xi = plsc.bitcast(x, jnp.int32)
    y = plsc.bitcast(jnp.int32(0x5F3759DF) - (xi >> 1), jnp.float32)
    for _ in range(3):
        y = y * (1.5 - 0.5 * x * y * y)
    return y


_MESH = plsc.VectorSubcoreMesh(core_axis_name="c", subcore_axis_name="s")


@functools.partial(
    pl.kernel,
    out_type=jax.ShapeDtypeStruct((2, NP, D_HID), jnp.float32),
    mesh=_MESH,
    compiler_params=pltpu.CompilerParams(
        use_tc_tiling_on_sc=False, needs_layout_passes=False),
    scratch_types=(
        pltpu.VMEM_SHARED((NP,), jnp.float32),
        pltpu.VMEM_SHARED((NP, D_HID), jnp.float32),
        pltpu.VMEM_SHARED((NP, D_HID), jnp.float32),
        pltpu.HBM((2, NP, D_HID), jnp.float32),
        pltpu.HBM((2, NP), jnp.float32),
        pltpu.VMEM((CONV_ROWS * 128,), jnp.int32),
        pltpu.VMEM((CONV_ROWS * 128,), jnp.float32),
        pltpu.VMEM((CONV_ROWS * 128,), jnp.int32),
        pltpu.VMEM((CONV_ROWS * 128,), jnp.float32),
        pltpu.VMEM((SLC, 128), jnp.int32),
        pltpu.VMEM((NP,), jnp.float32),
        pltpu.VMEM((RT,), jnp.float32),
        pltpu.VMEM((RT,), jnp.float32),
        pltpu.VMEM((128, D_HID), jnp.float32),
        pltpu.VMEM((128, D_HID), jnp.float32),
        pltpu.VMEM((128, D_HID), jnp.float32),
        pltpu.VMEM((128, D_HID), jnp.float32),
        pltpu.VMEM((128, D_HID), jnp.float32),
        pltpu.VMEM((RT, D_HID), jnp.float32),
        pltpu.VMEM((RT, D_HID), jnp.float32),
        pltpu.VMEM((16,), jnp.float32),
        pltpu.SemaphoreType.DMA,
        pltpu.SemaphoreType.DMA,
        pltpu.SemaphoreType.DMA,
        pltpu.SemaphoreType.DMA,
        pltpu.SemaphoreType.DMA,
        pltpu.SemaphoreType.DMA,
        pltpu.SemaphoreType.DMA,
        pltpu.SemaphoreType.DMA,
        pltpu.SemaphoreType.DMA,
        pltpu.SemaphoreType.REGULAR,
    ),
)
def _sc_gcn(ei_hbm, ew2_hbm, h_hbm, b1_hbm, part_hbm,
            deg_sp, tab_sp, agg_sp, p_hbm, dg_hbm, dstb, ewb, srcb, normb,
            idxb, dinvl, degl, degl2, rows0, rows1, rows2, rows3, zrows,
            pbuf0, pbuf1, b1b,
            gsem0, gsem1, gsem2, gsem3, ssem0, ssem1, ssem2, ssem3,
            dsem, bsem):
    c = lax.axis_index("c")
    s = lax.axis_index("s")
    w = c * 16 + s
    nbase = s * RT
    nsl = pl.ds(nbase, RT)

    bufs = (rows0, rows1, rows2, rows3)
    gsems = (gsem0, gsem1, gsem2, gsem3)
    ssems = (ssem0, ssem1, ssem2, ssem3)

    z16 = jnp.zeros((16,), jnp.float32)

    def zero_agg_slice():
        for k in range(SLC):
            pltpu.sync_copy(zrows, agg_sp.at[pl.ds(nbase + 128 * k, 128)])

    def scale_rows(rows, j):
        for g in range(8):
            nv = normb[pl.ds(128 * j + 16 * g, 16)]
            for l in range(16):
                e = 16 * g + l
                rows[e, :] = rows[e, :] * nv[l]

    def gather(b, j):
        pltpu.async_copy(tab_sp.at[srcb.at[pl.ds(128 * j, 128)]], bufs[b], gsems[b])

    def gather_wait(b, j):
        pltpu.make_async_copy(tab_sp.at[srcb.at[pl.ds(128 * j, 128)]], bufs[b], gsems[b]).wait()

    def scatter(b, j):
        pltpu.async_copy(bufs[b], agg_sp.at[dstb.at[pl.ds(128 * j, 128)]], ssems[b], add=True)

    def scatter_wait(b, j):
        pltpu.make_async_copy(bufs[b], agg_sp.at[dstb.at[pl.ds(128 * j, 128)]], ssems[b]).wait()

    def conv_pipeline():
        for b in range(NBUF):
            gather(b, b)

        def quad(jj, _):
            j = NBUF * jj
            for b in range(NBUF):
                gather_wait(b, j + b)
                scale_rows(bufs[b], j + b)
                scatter(b, j + b)
            for b in range(NBUF):
                scatter_wait(b, j + b)
                gather(b, j + b + NBUF)
            return 0

        lax.fori_loop(0, QUADS - 1, quad, 0)
        j = CONV_ROWS - NBUF
        for b in range(NBUF):
            gather_wait(b, j + b)
            scale_rows(bufs[b], j + b)
            scatter(b, j + b)
        for b in range(NBUF):
            scatter_wait(b, j + b)

    def selfloop_add():
        for k in range(SLC):
            pltpu.sync_copy(tab_sp.at[pl.ds(nbase + 128 * k, 128)], rows0)
            for g in range(8):
                dv = dinvl[pl.ds(nbase + 128 * k + 16 * g, 16)]
                dv2 = dv * dv
                for l in range(16):
                    e = 16 * g + l
                    rows0[e, :] = rows0[e, :] * dv2[l]
            pltpu.sync_copy(rows0, agg_sp.at[idxb.at[k]], add=True)

    for i in range(128):
        zrows[i, :] = z16
    zero_agg_slice()
    selfw = jnp.where(c == 0, jnp.float32(1.0), jnp.float32(0.0))
    initv = jnp.full((16,), 1.0, jnp.float32) * selfw
    for i in range(RT // 16):
        degl[pl.ds(16 * i, 16)] = initv
    pltpu.sync_copy(degl, deg_sp.at[nsl])
    lanes = lax.iota(jnp.int32, 16)
    for k in range(SLC):
        for g in range(8):
            idxb[k, pl.ds(16 * g, 16)] = nbase + 128 * k + 16 * g + lanes

    @pl.when(s == 15)
    def _():
        pltpu.sync_copy(h_hbm.at[pl.ds(15 * RT, LASTT)],
                        tab_sp.at[pl.ds(15 * RT, LASTT)])

    @pl.when(s < 15)
    def _():
        pltpu.sync_copy(h_hbm.at[nsl], tab_sp.at[nsl])

    pltpu.sync_copy(b1_hbm, b1b)
    plsc.subcore_barrier()

    dstart = (c * (EROWS // 2) + s * DEG_ROWS) * 128
    nrows_deg = jnp.where(s == 15, DEG_LAST, DEG_ROWS)

    @pl.when(s == 15)
    def _():
        off = (c * (EROWS // 2) + 15 * DEG_ROWS) * 128
        pltpu.sync_copy(ei_hbm.at[1, pl.ds(off, DEG_LAST * 128)],
                        dstb.at[pl.ds(0, DEG_LAST * 128)])
        pltpu.sync_copy(ew2_hbm.at[pl.ds(off, DEG_LAST * 128)],
                        ewb.at[pl.ds(0, DEG_LAST * 128)])

    @pl.when(s < 15)
    def _():
        pltpu.sync_copy(ei_hbm.at[1, pl.ds(dstart, DEG_ROWS * 128)],
                        dstb.at[pl.ds(0, DEG_ROWS * 128)])
        pltpu.sync_copy(ew2_hbm.at[pl.ds(dstart, DEG_ROWS * 128)],
                        ewb.at[pl.ds(0, DEG_ROWS * 128)])

    def deg_fire(i, _):
        pltpu.async_copy(ewb.at[pl.ds(128 * i, 128)], deg_sp.at[dstb.at[pl.ds(128 * i, 128)]], dsem, add=True)
        return 0

    def deg_drain(i, _):
        pltpu.make_async_copy(ewb.at[pl.ds(128 * i, 128)], deg_sp.at[dstb.at[pl.ds(128 * i, 128)]], dsem).wait()
        return 0

    lax.fori_loop(0, nrows_deg, deg_fire, 0)
    lax.fori_loop(0, nrows_deg, deg_drain, 0)
    plsc.subcore_barrier()

    pltpu.sync_copy(deg_sp.at[nsl], degl)
    pltpu.sync_copy(degl, dg_hbm.at[c, nsl])
    plsc.subcore_barrier()
    pltpu.core_barrier(bsem, core_axis_name="c")
    pltpu.sync_copy(dg_hbm.at[1 - c, nsl], degl2)
    for i in range(RT // 16):
        sl = pl.ds(16 * i, 16)
        x = jnp.maximum(degl[sl] + degl2[sl], 1.0)
        degl[sl] = _rsqrt_newton(x)
    pltpu.sync_copy(degl, deg_sp.at[nsl])
    plsc.subcore_barrier()
    pltpu.sync_copy(deg_sp, dinvl)

    estart = jnp.where(w == 31, EROWS - CONV_ROWS, w * CONV_ROWS) * 128
    esl = pl.ds(estart, CONV_ROWS * 128)
    pltpu.sync_copy(ei_hbm.at[0, esl], srcb)
    pltpu.sync_copy(ei_hbm.at[1, esl], dstb)
    pltpu.sync_copy(ew2_hbm.at[esl], ewb)

    def norm_row(j, _):
        for g in range(8):
            sl = pl.ds(128 * j + 16 * g, 16)
            nrm = (plsc.load_gather(dinvl, [srcb[sl]]) * ewb[sl]
                   * plsc.load_gather(dinvl, [dstb[sl]]))
            normb[sl] = nrm
        return 0

    lax.fori_loop(0, CONV_ROWS, norm_row, 0)

    @pl.when(w == 30)
    def _():
        for j in range((EROWS - CONV_ROWS) - 30 * CONV_ROWS, CONV_ROWS):
            for g in range(8):
                normb[pl.ds(128 * j + 16 * g, 16)] = z16

    conv_pipeline()

    @pl.when(c == 0)
    def _():
        selfloop_add()

    plsc.subcore_barrier()
    pltpu.sync_copy(agg_sp.at[nsl], p_hbm.at[c, nsl])
    zero_agg_slice()
    plsc.subcore_barrier()
    pltpu.core_barrier(bsem, core_axis_name="c")

    pltpu.sync_copy(p_hbm.at[0, nsl], pbuf0)
    pltpu.sync_copy(p_hbm.at[1, nsl], pbuf1)
    b1v = b1b[:]

    def relu_step(ii, _):
        for r in range(16):
            i = 16 * ii + r
            pbuf0[i, :] = jnp.maximum(pbuf0[i, :] + pbuf1[i, :] + b1v, 0.0)
        return 0

    lax.fori_loop(0, RT // 16, relu_step, 0)
    pltpu.sync_copy(pbuf0, tab_sp.at[nsl])
    plsc.subcore_barrier()

    conv_pipeline()

    @pl.when(c == 0)
    def _():
        selfloop_add()

    plsc.subcore_barrier()
    pltpu.sync_copy(agg_sp.at[nsl], part_hbm.at[c, nsl])


def _tc_matmul_body(x_ref, w_ref, o_ref):
    o_ref[:, :] = jnp.dot(x_ref[:, :], w_ref[:, :],
                          preferred_element_type=jnp.float32)


def _tc_matmul(x, w):
    return pl.pallas_call(
        _tc_matmul_body,
        grid=(N_NODES // MB,),
        in_specs=[
            pl.BlockSpec((MB, D_FEAT), lambda i: (i, 0)),
            pl.BlockSpec((D_FEAT, D_HID), lambda i: (0, 0)),
        ],
        out_specs=pl.BlockSpec((MB, D_HID), lambda i: (i, 0)),
        out_shape=jax.ShapeDtypeStruct((N_NODES, D_HID), jnp.float32),
    )(x, w)


def _tc_finish_body(q_ref, w_ref, b_ref, o_ref):
    q = q_ref[0] + q_ref[1]
    t = jnp.dot(q, w_ref[:, :], preferred_element_type=jnp.float32) + b_ref[:, :]
    m = jnp.max(t, axis=1, keepdims=True)
    e = jnp.exp(t - m)
    lse = jnp.log(jnp.sum(e, axis=1, keepdims=True))
    o_ref[:, :] = t - m - lse


def _tc_finish(q, w2, b2):
    return pl.pallas_call(
        _tc_finish_body,
        grid=(N_NODES // FB,),
        in_specs=[
            pl.BlockSpec((2, FB, D_HID), lambda i: (0, i, 0)),
            pl.BlockSpec((D_HID, N_CLASS), lambda i: (0, 0)),
            pl.BlockSpec((1, N_CLASS), lambda i: (0, 0)),
        ],
        out_specs=pl.BlockSpec((FB, N_CLASS), lambda i: (i, 0)),
        out_shape=jax.ShapeDtypeStruct((N_NODES, N_CLASS), jnp.float32),
    )(q, w2, b2)


def kernel(x, edge_index, edge_attr, W1, b1, W2, b2):
    ei = edge_index.astype(jnp.int32)
    h1 = _tc_matmul(x, W1)
    part2 = _sc_gcn(ei, edge_attr, h1, b1)
    return _tc_finish(part2, W2, b2.reshape(1, N_CLASS))


# speedup vs baseline: 84.2818x; 1.0975x over previous
import functools

import jax
import jax.numpy as jnp
from jax import lax
from jax.experimental import pallas as pl
from jax.experimental.pallas import tpu as pltpu
from jax.experimental.pallas import tpu_sc as plsc

N_NODES = 10000
D_FEAT = 128
D_HID = 16
N_CLASS = 128

NP = 10240
RT = NP // 16
LASTT = N_NODES - 15 * RT
EROWS = 2500
DEG_ROWS = 79
DEG_LAST = EROWS // 2 - 15 * DEG_ROWS
CONV_ROWS = 80
QUADS = CONV_ROWS // 4
NBUF = 4
SLC = RT // 128
MB = 2000
FB = 2048


def _rsqrt_newton(x):
    xi = plsc.bitcast(x, jnp.int32)
    y = plsc.bitcast(jnp.int32(0x5F3759DF) - (xi >> 1), jnp.float32)
    for _ in range(3):
        y = y * (1.5 - 0.5 * x * y * y)
    return y


_MESH = plsc.VectorSubcoreMesh(core_axis_name="c", subcore_axis_name="s")


@functools.partial(
    pl.kernel,
    out_type=jax.ShapeDtypeStruct((2, NP, D_HID), jnp.float32),
    mesh=_MESH,
    compiler_params=pltpu.CompilerParams(
        use_tc_tiling_on_sc=False, needs_layout_passes=False),
    scratch_types=(
        pltpu.VMEM_SHARED((NP,), jnp.float32),
        pltpu.VMEM_SHARED((NP, D_HID), jnp.float32),
        pltpu.VMEM_SHARED((NP, D_HID), jnp.float32),
        pltpu.HBM((2, NP, D_HID), jnp.float32),
        pltpu.HBM((2, NP), jnp.float32),
        pltpu.VMEM((CONV_ROWS * 128,), jnp.int32),
        pltpu.VMEM((CONV_ROWS * 128,), jnp.float32),
        pltpu.VMEM((CONV_ROWS * 128,), jnp.int32),
        pltpu.VMEM((CONV_ROWS * 128,), jnp.float32),
        pltpu.VMEM((SLC, 128), jnp.int32),
        pltpu.VMEM((NP,), jnp.float32),
        pltpu.VMEM((RT,), jnp.float32),
        pltpu.VMEM((RT,), jnp.float32),
        pltpu.VMEM((128, D_HID), jnp.float32),
        pltpu.VMEM((128, D_HID), jnp.float32),
        pltpu.VMEM((128, D_HID), jnp.float32),
        pltpu.VMEM((128, D_HID), jnp.float32),
        pltpu.VMEM((128, D_HID), jnp.float32),
        pltpu.VMEM((RT, D_HID), jnp.float32),
        pltpu.VMEM((RT, D_HID), jnp.float32),
        pltpu.VMEM((16,), jnp.float32),
        pltpu.SemaphoreType.DMA,
        pltpu.SemaphoreType.DMA,
        pltpu.SemaphoreType.DMA,
        pltpu.SemaphoreType.DMA,
        pltpu.SemaphoreType.DMA,
        pltpu.SemaphoreType.DMA,
        pltpu.SemaphoreType.DMA,
        pltpu.SemaphoreType.DMA,
        pltpu.SemaphoreType.DMA,
        pltpu.SemaphoreType.REGULAR,
    ),
)
def _sc_gcn(ei_hbm, ew2_hbm, h_hbm, b1_hbm, part_hbm,
            deg_sp, tab_sp, agg_sp, p_hbm, dg_hbm, dstb, ewb, srcb, normb,
            idxb, dinvl, degl, degl2, rows0, rows1, rows2, rows3, zrows,
            pbuf0, pbuf1, b1b,
            gsem0, gsem1, gsem2, gsem3, ssem0, ssem1, ssem2, ssem3,
            dsem, bsem):
    c = lax.axis_index("c")
    s = lax.axis_index("s")
    w = c * 16 + s
    nbase = s * RT
    nsl = pl.ds(nbase, RT)

    bufs = (rows0, rows1, rows2, rows3)
    gsems = (gsem0, gsem1, gsem2, gsem3)
    ssems = (ssem0, ssem1, ssem2, ssem3)

    z16 = jnp.zeros((16,), jnp.float32)

    def zero_agg_slice():
        for k in range(SLC):
            pltpu.sync_copy(zrows, agg_sp.at[pl.ds(nbase + 128 * k, 128)])

    def scale_rows(rows, j):
        for g in range(8):
            nv = normb[pl.ds(128 * j + 16 * g, 16)]
            for l in range(16):
                e = 16 * g + l
                rows[e, :] = rows[e, :] * nv[l]

    def gather(b, j):
        pltpu.async_copy(tab_sp.at[srcb.at[pl.ds(128 * j, 128)]], bufs[b], gsems[b])

    def gather_wait(b, j):
        pltpu.make_async_copy(tab_sp.at[srcb.at[pl.ds(128 * j, 128)]], bufs[b], gsems[b]).wait()

    def scatter(b, j):
        pltpu.async_copy(bufs[b], agg_sp.at[dstb.at[pl.ds(128 * j, 128)]], ssems[b], add=True)

    def scatter_wait(b, j):
        pltpu.make_async_copy(bufs[b], agg_sp.at[dstb.at[pl.ds(128 * j, 128)]], ssems[b]).wait()

    def conv_pipeline():
        for b in range(NBUF):
            gather(b, b)

        def quad(jj, _):
            j = NBUF * jj
            for b in range(NBUF):
                gather_wait(b, j + b)
                scale_rows(bufs[b], j + b)
                scatter(b, j + b)
            for b in range(NBUF):
                scatter_wait(b, j + b)
                gather(b, j + b + NBUF)
            return 0

        lax.fori_loop(0, QUADS - 1, quad, 0)
        j = CONV_ROWS - NBUF
        for b in range(NBUF):
            gather_wait(b, j + b)
            scale_rows(bufs[b], j + b)
            scatter(b, j + b)
        for b in range(NBUF):
            scatter_wait(b, j + b)

    def selfloop_add():
        for k in range(SLC):
            pltpu.sync_copy(tab_sp.at[pl.ds(nbase + 128 * k, 128)], rows0)
            for g in range(8):
                dv = dinvl[pl.ds(nbase + 128 * k + 16 * g, 16)]
                dv2 = dv * dv
                for l in range(16):
                    e = 16 * g + l
                    rows0[e, :] = rows0[e, :] * dv2[l]
            pltpu.sync_copy(rows0, agg_sp.at[idxb.at[k]], add=True)

    for i in range(128):
        zrows[i, :] = z16
    zero_agg_slice()
    selfw = jnp.where(c == 0, jnp.float32(1.0), jnp.float32(0.0))
    initv = jnp.full((16,), 1.0, jnp.float32) * selfw
    for i in range(RT // 16):
        degl[pl.ds(16 * i, 16)] = initv
    pltpu.sync_copy(degl, deg_sp.at[nsl])
    lanes = lax.iota(jnp.int32, 16)
    for k in range(SLC):
        for g in range(8):
            idxb[k, pl.ds(16 * g, 16)] = nbase + 128 * k + 16 * g + lanes

    @pl.when(s == 15)
    def _():
        pltpu.sync_copy(h_hbm.at[pl.ds(15 * RT, LASTT)],
                        tab_sp.at[pl.ds(15 * RT, LASTT)])

    @pl.when(s < 15)
    def _():
        pltpu.sync_copy(h_hbm.at[nsl], tab_sp.at[nsl])

    pltpu.sync_copy(b1_hbm, b1b)
    plsc.subcore_barrier()

    dstart = (c * (EROWS // 2) + s * DEG_ROWS) * 128
    nrows_deg = jnp.where(s == 15, DEG_LAST, DEG_ROWS)

    @pl.when(s == 15)
    def _():
        off = (c * (EROWS // 2) + 15 * DEG_ROWS) * 128
        pltpu.sync_copy(ei_hbm.at[1, pl.ds(off, DEG_LAST * 128)],
                        dstb.at[pl.ds(0, DEG_LAST * 128)])
        pltpu.sync_copy(ew2_hbm.at[pl.ds(off, DEG_LAST * 128)],
                        ewb.at[pl.ds(0, DEG_LAST * 128)])

    @pl.when(s < 15)
    def _():
        pltpu.sync_copy(ei_hbm.at[1, pl.ds(dstart, DEG_ROWS * 128)],
                        dstb.at[pl.ds(0, DEG_ROWS * 128)])
        pltpu.sync_copy(ew2_hbm.at[pl.ds(dstart, DEG_ROWS * 128)],
                        ewb.at[pl.ds(0, DEG_ROWS * 128)])

    def deg_fire(i, _):
        pltpu.async_copy(ewb.at[pl.ds(128 * i, 128)], deg_sp.at[dstb.at[pl.ds(128 * i, 128)]], dsem, add=True)
        return 0

    def deg_drain(i, _):
        pltpu.make_async_copy(ewb.at[pl.ds(128 * i, 128)], deg_sp.at[dstb.at[pl.ds(128 * i, 128)]], dsem).wait()
        return 0

    lax.fori_loop(0, nrows_deg, deg_fire, 0)
    lax.fori_loop(0, nrows_deg, deg_drain, 0)
    plsc.subcore_barrier()

    pltpu.sync_copy(deg_sp.at[nsl], degl)
    pltpu.sync_copy(degl, dg_hbm.at[c, nsl])
    plsc.subcore_barrier()
    pltpu.core_barrier(bsem, core_axis_name="c")
    pltpu.sync_copy(dg_hbm.at[1 - c, nsl], degl2)
    for i in range(RT // 16):
        sl = pl.ds(16 * i, 16)
        x = jnp.maximum(degl[sl] + degl2[sl], 1.0)
        degl[sl] = _rsqrt_newton(x)
    pltpu.sync_copy(degl, deg_sp.at[nsl])
    plsc.subcore_barrier()
    pltpu.sync_copy(deg_sp, dinvl)

    estart = jnp.where(w == 31, EROWS - CONV_ROWS, w * CONV_ROWS) * 128
    esl = pl.ds(estart, CONV_ROWS * 128)
    pltpu.sync_copy(ei_hbm.at[0, esl], srcb)
    pltpu.sync_copy(ei_hbm.at[1, esl], dstb)
    pltpu.sync_copy(ew2_hbm.at[esl], ewb)

    def norm_row(j, _):
        for g in range(8):
            sl = pl.ds(128 * j + 16 * g, 16)
            nrm = (plsc.load_gather(dinvl, [srcb[sl]]) * ewb[sl]
                   * plsc.load_gather(dinvl, [dstb[sl]]))
            normb[sl] = nrm
        return 0

    lax.fori_loop(0, CONV_ROWS, norm_row, 0)

    @pl.when(w == 30)
    def _():
        for j in range((EROWS - CONV_ROWS) - 30 * CONV_ROWS, CONV_ROWS):
            for g in range(8):
                normb[pl.ds(128 * j + 16 * g, 16)] = z16

    conv_pipeline()

    @pl.when(c == 0)
    def _():
        selfloop_add()

    plsc.subcore_barrier()
    pltpu.sync_copy(agg_sp.at[nsl], p_hbm.at[c, nsl])
    zero_agg_slice()
    plsc.subcore_barrier()
    pltpu.core_barrier(bsem, core_axis_name="c")

    pltpu.sync_copy(p_hbm.at[0, nsl], pbuf0)
    pltpu.sync_copy(p_hbm.at[1, nsl], pbuf1)
    b1v = b1b[:]

    def relu_step(ii, _):
        for r in range(16):
            i = 16 * ii + r
            pbuf0[i, :] = jnp.maximum(pbuf0[i, :] + pbuf1[i, :] + b1v, 0.0)
        return 0

    lax.fori_loop(0, RT // 16, relu_step, 0)
    pltpu.sync_copy(pbuf0, tab_sp.at[nsl])
    plsc.subcore_barrier()

    conv_pipeline()

    @pl.when(c == 0)
    def _():
        selfloop_add()

    plsc.subcore_barrier()
    pltpu.sync_copy(agg_sp.at[nsl], part_hbm.at[c, nsl])


def _tc_matmul_body(x_ref, w_ref, o_ref):
    xb = x_ref[:, :].reshape(-1, 8, D_FEAT)
    parts = [jnp.dot(xb[:, k, :], w_ref[:, :], preferred_element_type=jnp.float32)
             for k in range(8)]
    o_ref[:, :] = jnp.concatenate(parts, axis=1)


def _tc_matmul(x, w):
    nblk = 2048
    oblk = nblk * D_HID // 128
    return pl.pallas_call(
        _tc_matmul_body,
        grid=(5,),
        in_specs=[
            pl.BlockSpec((nblk, D_FEAT), lambda i: (i, 0)),
            pl.BlockSpec((D_FEAT, D_HID), lambda i: (0, 0)),
        ],
        out_specs=pl.BlockSpec((oblk, 128), lambda i: (i, 0)),
        out_shape=jax.ShapeDtypeStruct((NP * D_HID // 128, 128), jnp.float32),
    )(x, w)


def _tc_finish_body(q_ref, w_ref, b_ref, o_ref):
    qq = q_ref[:, :, :]
    qsum = qq[0] + qq[1]
    parts = [jnp.dot(qsum[:, 16 * k:16 * (k + 1)], w_ref[:, :],
                     preferred_element_type=jnp.float32)
             for k in range(8)]
    t = jnp.stack(parts, axis=1).reshape(FB, N_CLASS) + b_ref[:, :]
    m = jnp.max(t, axis=1, keepdims=True)
    e = jnp.exp(t - m)
    lse = jnp.log(jnp.sum(e, axis=1, keepdims=True))
    o_ref[:, :] = t - m - lse


def _tc_finish(q, w2, b2):
    return pl.pallas_call(
        _tc_finish_body,
        grid=(5,),
        in_specs=[
            pl.BlockSpec((2, FB * D_HID // 128, 128), lambda i: (0, i, 0)),
            pl.BlockSpec((D_HID, N_CLASS), lambda i: (0, 0)),
            pl.BlockSpec((1, N_CLASS), lambda i: (0, 0)),
        ],
        out_specs=pl.BlockSpec((FB, N_CLASS), lambda i: (i, 0)),
        out_shape=jax.ShapeDtypeStruct((N_NODES, N_CLASS), jnp.float32),
    )(q, w2, b2)


def kernel(x, edge_index, edge_attr, W1, b1, W2, b2):
    ei = edge_index.astype(jnp.int32)
    h1 = _tc_matmul(x, W1).reshape(NP, D_HID)
    part2 = _sc_gcn(ei, edge_attr, h1, b1)
    q = part2.reshape(2, NP * D_HID // 128, 128)
    return _tc_finish(q, W2, b2.reshape(1, N_CLASS))
